# baseline (speedup 1.0000x reference)
import functools

import jax
import jax.numpy as jnp
from jax import lax
from jax.experimental import pallas as pl
from jax.experimental.pallas import tpu as pltpu
from jax.experimental.pallas import tpu_sc as plsc

HEAD_DIM = 128
CHUNK = 64
NBUF = 4


def _rope_gather_fn(N, chunks_per_w, NC):
    mesh = plsc.VectorSubcoreMesh(core_axis_name="c", subcore_axis_name="s")

    @functools.partial(
        pl.kernel,
        mesh=mesh,
        out_type=(
            jax.ShapeDtypeStruct((N, HEAD_DIM), jnp.float32),
            jax.ShapeDtypeStruct((N, HEAD_DIM), jnp.float32),
        ),
        scratch_types=[
            pltpu.VMEM((chunks_per_w, CHUNK), jnp.int32),
            pltpu.VMEM((NBUF, CHUNK, HEAD_DIM), jnp.float32),
            pltpu.VMEM((NBUF, CHUNK, HEAD_DIM), jnp.float32),
        ]
        + [pltpu.SemaphoreType.DMA] * (2 * NBUF),
    )
    def body(idx_hbm, cos_hbm, sin_hbm, cos_out, sin_out,
             idx_v, cos_v, sin_v, *sems):
        gsem, wsem = sems[:NBUF], sems[NBUF:]
        wid = lax.axis_index("s") * NC + lax.axis_index("c")
        row0 = wid * chunks_per_w
        pltpu.sync_copy(idx_hbm.at[pl.ds(row0, chunks_per_w)], idx_v)

        def issue_gather(j):
            b = j % NBUF
            return (
                pltpu.async_copy(cos_hbm.at[idx_v.at[j]], cos_v.at[b], gsem[b]),
                pltpu.async_copy(sin_hbm.at[idx_v.at[j]], sin_v.at[b], gsem[b]),
            )

        pending_g = [None] * NBUF
        pending_w = [None] * NBUF
        for j in range(min(NBUF - 1, chunks_per_w)):
            pending_g[j % NBUF] = issue_gather(j)
        for j in range(chunks_per_w):
            b = j % NBUF
            jn = j + NBUF - 1
            if jn < chunks_per_w:
                nb = jn % NBUF
                if pending_w[nb] is not None:
                    for d in pending_w[nb]:
                        d.wait()
                    pending_w[nb] = None
                pending_g[nb] = issue_gather(jn)
            for d in pending_g[b]:
                d.wait()
            pending_g[b] = None
            base = (row0 + j) * CHUNK
            pending_w[b] = (
                pltpu.async_copy(cos_v.at[b], cos_out.at[pl.ds(base, CHUNK)],
                                 wsem[b]),
                pltpu.async_copy(sin_v.at[b], sin_out.at[pl.ds(base, CHUNK)],
                                 wsem[b]),
            )
        for w in pending_w:
            if w is not None:
                for d in w:
                    d.wait()

    return body


def kernel(position_ids, cos_emb, sin_emb):
    B, S = position_ids.shape
    N = B * S
    info = plsc.get_sparse_core_info()
    NC, NS = info.num_cores, info.num_subcores
    NW = NC * NS
    chunks_total = N // CHUNK
    chunks_per_w = chunks_total // NW

    idx = position_ids.reshape(chunks_total, CHUNK).astype(jnp.int32)
    cos_flat, sin_flat = _rope_gather_fn(N, chunks_per_w, NC)(
        idx, cos_emb, sin_emb)
    return (cos_flat.reshape(B, S, HEAD_DIM),
            sin_flat.reshape(B, S, HEAD_DIM))


# speedup vs baseline: 1.0263x; 1.0263x over previous
import functools

import jax
import jax.numpy as jnp
from jax import lax
from jax.experimental import pallas as pl
from jax.experimental.pallas import tpu as pltpu
from jax.experimental.pallas import tpu_sc as plsc

HEAD_DIM = 128
CHUNK = 128
NBUF = 3


def _rope_gather_fn(B, S, NC, NS):
    mesh = plsc.VectorSubcoreMesh(core_axis_name="c", subcore_axis_name="s")
    N = B * S
    NW = NC * NS
    per_w = N // NW
    blocks = S // per_w
    chunks_per_w = per_w // CHUNK

    @functools.partial(
        pl.kernel,
        mesh=mesh,
        out_type=(
            jax.ShapeDtypeStruct((N, HEAD_DIM), jnp.float32),
            jax.ShapeDtypeStruct((N, HEAD_DIM), jnp.float32),
        ),
        scratch_types=[
            pltpu.VMEM((per_w,), jnp.int32),
            pltpu.VMEM((NBUF, CHUNK, HEAD_DIM), jnp.float32),
            pltpu.VMEM((NBUF, CHUNK, HEAD_DIM), jnp.float32),
        ]
        + [pltpu.SemaphoreType.DMA] * (2 * NBUF),
    )
    def body(idx_hbm, cos_hbm, sin_hbm, cos_out, sin_out,
             idx_v, cos_v, sin_v, *sems):
        gsem, wsem = sems[:NBUF], sems[NBUF:]
        wid = lax.axis_index("s") * NC + lax.axis_index("c")
        batch = wid // blocks
        col0 = (wid % blocks) * per_w
        row0 = wid * per_w
        pltpu.sync_copy(idx_hbm.at[batch, pl.ds(col0, per_w)], idx_v)

        def issue_gather(j):
            b = j % NBUF
            ids = idx_v.at[pl.ds(j * CHUNK, CHUNK)]
            return (
                pltpu.async_copy(cos_hbm.at[ids], cos_v.at[b], gsem[b]),
                pltpu.async_copy(sin_hbm.at[ids], sin_v.at[b], gsem[b]),
            )

        pending_g = [None] * NBUF
        pending_w = [None] * NBUF
        for j in range(min(NBUF - 1, chunks_per_w)):
            pending_g[j % NBUF] = issue_gather(j)
        for j in range(chunks_per_w):
            b = j % NBUF
            jn = j + NBUF - 1
            if jn < chunks_per_w:
                nb = jn % NBUF
                if pending_w[nb] is not None:
                    for d in pending_w[nb]:
                        d.wait()
                    pending_w[nb] = None
                pending_g[nb] = issue_gather(jn)
            for d in pending_g[b]:
                d.wait()
            pending_g[b] = None
            base = row0 + j * CHUNK
            pending_w[b] = (
                pltpu.async_copy(cos_v.at[b], cos_out.at[pl.ds(base, CHUNK)],
                                 wsem[b]),
                pltpu.async_copy(sin_v.at[b], sin_out.at[pl.ds(base, CHUNK)],
                                 wsem[b]),
            )
        for w in pending_w:
            if w is not None:
                for d in w:
                    d.wait()

    return body


def kernel(position_ids, cos_emb, sin_emb):
    B, S = position_ids.shape
    info = plsc.get_sparse_core_info()
    NC, NS = info.num_cores, info.num_subcores
    idx = position_ids.astype(jnp.int32)
    cos_flat, sin_flat = _rope_gather_fn(B, S, NC, NS)(idx, cos_emb, sin_emb)
    return (cos_flat.reshape(B, S, HEAD_DIM),
            sin_flat.reshape(B, S, HEAD_DIM))
